# baseline (speedup 1.0000x reference)
import functools

import jax
import jax.numpy as jnp
from jax import lax
from jax.experimental import pallas as pl
from jax.experimental.pallas import tpu as pltpu
from jax.experimental.pallas import tpu_sc as plsc

N = 10000
C = 128
K = 16
KC = 32
NP = 10240
RK = 80
RC = 200
INF = float("inf")


def _knn_body(pos8_ref, posT8_ref, out_ref, dist_ref, dm_ref):
    pid = pl.program_id(0)
    prow = pos8_ref[...]
    pcol = posT8_ref[...]
    d2r = jnp.sum(prow * prow, axis=1, keepdims=True)
    d2c = jnp.sum(pcol * pcol, axis=0, keepdims=True)
    cross = jnp.dot(prow, pcol, preferred_element_type=jnp.float32)
    dist = d2r + d2c - 2.0 * cross
    colio = lax.broadcasted_iota(jnp.int32, (RK, NP), 1)
    rowio = lax.broadcasted_iota(jnp.int32, (RK, NP), 0) + pid * RK
    dist_ref[...] = jnp.where((colio == rowio) | (colio >= N), INF, dist)

    nsub = NP // 128
    lane128 = lax.broadcasted_iota(jnp.int32, (RK, 128), 1)
    subio = lax.broadcasted_iota(jnp.int32, (RK, nsub, 128), 1)
    NR = KC - 1
    BIGCOL = 1 << 30

    def round_body(carry):
        T, cT, cnt, acc = carry
        d = dist_ref[...]
        dm2 = jnp.where((d < T) | ((d == T) & (colio <= cT)), INF, d)
        dm_ref[...] = dm2
        dm = dm_ref[...].reshape(RK, nsub, 128)

        def lane_top(excl):
            m = jnp.min(jnp.where(excl, INF, dm), axis=1)
            a = jnp.min(
                jnp.where((dm == m[:, None, :]) & ~excl, subio, nsub),
                axis=1)
            return m, a, excl | (subio == a[:, None, :])

        no = jnp.zeros((RK, nsub, 128), jnp.bool_)
        v1, a1, e1 = lane_top(no)
        v2, a2, e2 = lane_top(e1)
        v3, a3, e3 = lane_top(e2)
        v4, a4, _ = lane_top(e3)
        c1 = a1 * 128 + lane128
        c2 = a2 * 128 + lane128
        c3 = a3 * 128 + lane128
        c4 = a4 * 128 + lane128

        def mstep(t, st):
            cur, curcol, currank, T, cT, cnt, acc, active = st
            gm = jnp.min(cur, axis=1, keepdims=True)
            gi = jnp.min(jnp.where(cur == gm, curcol, BIGCOL),
                         axis=1, keepdims=True)
            valid = (active > 0) & (cnt < NR) & (gm < INF)
            vsel = (curcol == gi) & valid
            acc = jnp.where(valid & (cnt < K) & (lane128 == cnt), gi, acc)
            acc = jnp.where(
                valid & (cnt % 2 == 0) & (lane128 == K + cnt // 2), gi, acc)
            T = jnp.where(valid, gm, T)
            cT = jnp.where(valid, gi, cT)
            ex = jnp.max(jnp.where(vsel & (currank == 3), 1, 0),
                         axis=1, keepdims=True)
            active = jnp.where(ex > 0, 0, active)
            nv = jnp.where(currank == 0, v2,
                           jnp.where(currank == 1, v3,
                                     jnp.where(currank == 2, v4, INF)))
            nc = jnp.where(currank == 0, c2,
                           jnp.where(currank == 1, c3,
                                     jnp.where(currank == 2, c4, BIGCOL)))
            cur = jnp.where(vsel, nv, cur)
            curcol = jnp.where(vsel, nc, curcol)
            currank = jnp.where(vsel, currank + 1, currank)
            cnt = cnt + jnp.where(valid, 1, 0)
            return cur, curcol, currank, T, cT, cnt, acc, active

        st = (v1, c1, jnp.zeros((RK, 128), jnp.int32), T, cT, cnt, acc,
              jnp.ones((RK, 1), jnp.int32))
        st = lax.fori_loop(0, NR, mstep, st)
        return st[3], st[4], st[5], st[6]

    T0 = jnp.full((RK, 1), -INF, jnp.float32)
    c0 = jnp.full((RK, 1), -1, jnp.int32)
    carry = (T0, c0, jnp.zeros((RK, 1), jnp.int32),
             jnp.zeros((RK, 128), jnp.int32))
    carry = lax.while_loop(lambda c: jnp.min(c[2]) < NR, round_body, carry)
    out_ref[...] = carry[3][:, :KC]


def _knn(pos8, posT8):
    return pl.pallas_call(
        _knn_body,
        grid=(N // RK,),
        in_specs=[
            pl.BlockSpec((RK, 8), lambda i: (i, 0)),
            pl.BlockSpec((8, NP), lambda i: (0, 0)),
        ],
        out_specs=pl.BlockSpec((RK, KC), lambda i: (i, 0)),
        out_shape=jax.ShapeDtypeStruct((N, KC), jnp.int32),
        scratch_shapes=[pltpu.VMEM((RK, NP), jnp.float32),
                        pltpu.VMEM((RK, NP), jnp.float32)],
    )(pos8, posT8)


def _gather_sc(x, pos16, idx_flat):
    info = plsc.get_sparse_core_info()
    nc, ns = info.num_cores, info.num_subcores
    nw = nc * ns
    b_per_w = (N * KC) // nw
    ch = 128
    nfull = b_per_w // ch
    tail = b_per_w - nfull * ch

    mesh = plsc.VectorSubcoreMesh(core_axis_name="c", subcore_axis_name="s")

    @functools.partial(
        pl.kernel, mesh=mesh,
        compiler_params=pltpu.CompilerParams(use_tc_tiling_on_sc=False),
        out_type=(
            jax.ShapeDtypeStruct((N * KC, C), jnp.float32),
            jax.ShapeDtypeStruct((N * KC, 16), jnp.float32),
        ),
        scratch_types=[
            pltpu.VMEM((ch,), jnp.int32),
            pltpu.VMEM((ch, C), jnp.float32),
            pltpu.VMEM((ch, 16), jnp.float32),
            pltpu.SemaphoreType.DMA,
            pltpu.SemaphoreType.DMA,
        ],
    )
    def gk(x_hbm, p_hbm, idx_hbm, xg_hbm, pg_hbm, idx_v, xrow_v, prow_v,
           sem1, sem2):
        wid = lax.axis_index("s") * nc + lax.axis_index("c")
        base_w = wid * b_per_w

        def run_chunk(base, cnt):
            pltpu.sync_copy(idx_hbm.at[pl.ds(base, cnt)],
                            idx_v.at[pl.ds(0, cnt)])
            cp1 = pltpu.async_copy(x_hbm.at[idx_v.at[pl.ds(0, cnt)]],
                                   xrow_v.at[pl.ds(0, cnt)], sem1)
            cp2 = pltpu.async_copy(p_hbm.at[idx_v.at[pl.ds(0, cnt)]],
                                   prow_v.at[pl.ds(0, cnt)], sem2)
            cp1.wait()
            cp2.wait()
            pltpu.sync_copy(xrow_v.at[pl.ds(0, cnt)],
                            xg_hbm.at[pl.ds(base, cnt)])
            pltpu.sync_copy(prow_v.at[pl.ds(0, cnt)],
                            pg_hbm.at[pl.ds(base, cnt)])

        def body(c, carry):
            run_chunk(base_w + c * ch, ch)
            return carry

        lax.fori_loop(0, nfull, body, 0)
        run_chunk(base_w + nfull * ch, tail)

    return gk(x, pos16, idx_flat)


def _leaky(x):
    return jnp.where(x >= 0, x, 0.2 * x)


def _conv_body(x_ref, pos_ref, xg_ref, pg_ref,
               wl0_ref, bl0_ref, ws0_ref, bs0_ref, wd0_ref, bd0_ref,
               wp0_ref, bp0_ref,
               wl1_ref, bl1_ref, ws1_ref, bs1_ref, wd1_ref, bd1_ref,
               wp1_ref, bp1_ref,
               wgx_ref, wgp_ref, bg_ref, wo1_ref, bo1_ref, wo2_ref, bo2_ref,
               out_ref):
    xb = x_ref[...]
    posb = pos_ref[...]
    xg3 = xg_ref[...].reshape(RC, KC, C)
    pg3 = pg_ref[...].reshape(RC, KC, 16)

    h = _leaky(jnp.dot(xb, wgx_ref[...], preferred_element_type=jnp.float32)
               + jnp.dot(posb, wgp_ref[...],
                         preferred_element_type=jnp.float32)
               + bg_ref[...])

    layer = [
        (wl0_ref, bl0_ref, ws0_ref, bs0_ref, wd0_ref, bd0_ref, wp0_ref,
         bp0_ref),
        (wl1_ref, bl1_ref, ws1_ref, bs1_ref, wd1_ref, bd1_ref, wp1_ref,
         bp1_ref),
    ]
    for li, (wl, bl, ws, bs, wd, bd, wp, bp) in enumerate(layer):
        xgl = xg3[:, li * K:(li + 1) * K, :].reshape(RC * K, C)
        pgl = pg3[:, li * K:(li + 1) * K, :]
        v = jnp.dot(xgl, wl[...], preferred_element_type=jnp.float32) + bl[...]
        asrc = (jnp.dot(xgl, ws[...], preferred_element_type=jnp.float32)
                + bs[...])
        adst = (jnp.dot(xb, wd[...], preferred_element_type=jnp.float32)
                + bd[...])
        rel = (posb[:, None, :] - pgl).reshape(RC * K, 16)
        delta = (jnp.dot(rel, wp[...], preferred_element_type=jnp.float32)
                 + bp[...])
        delta3 = delta.reshape(RC, K, C)
        alpha = adst[:, None, :] - asrc.reshape(RC, K, C) + delta3
        mx = jnp.max(alpha, axis=1, keepdims=True)
        e = jnp.exp(alpha - mx)
        p = e / jnp.sum(e, axis=1, keepdims=True)
        msg = p * (v.reshape(RC, K, C) + delta3)
        h = h + jnp.sum(msg, axis=1)

    res = (jnp.dot(_leaky(jnp.dot(h, wo1_ref[...],
                                  preferred_element_type=jnp.float32)
                          + bo1_ref[...]),
                   wo2_ref[...], preferred_element_type=jnp.float32)
           + bo2_ref[...])
    out_ref[...] = res + posb


def _conv(x, pos16, xg, pg, weights):
    full = lambda a: pl.BlockSpec(a.shape, lambda i: tuple(0 for _ in a.shape))
    in_specs = [
        pl.BlockSpec((RC, C), lambda i: (i, 0)),
        pl.BlockSpec((RC, 16), lambda i: (i, 0)),
        pl.BlockSpec((RC * KC, C), lambda i: (i, 0)),
        pl.BlockSpec((RC * KC, 16), lambda i: (i, 0)),
    ] + [full(w) for w in weights]
    return pl.pallas_call(
        _conv_body,
        grid=(N // RC,),
        in_specs=in_specs,
        out_specs=pl.BlockSpec((RC, 16), lambda i: (i, 0)),
        out_shape=jax.ShapeDtypeStruct((N, 16), jnp.float32),
    )(x, pos16, xg, pg, *weights)


def kernel(x, pos,
           W_lin0, b_lin0, W_src0, b_src0, W_dst0, b_dst0, W_pos0, b_pos0,
           W_lin1, b_lin1, W_src1, b_src1, W_dst1, b_dst1, W_pos1, b_pos1,
           W_g, b_g, W_o1, b_o1, W_o2, b_o2):
    pos16 = jnp.zeros((N, 16), jnp.float32).at[:, :3].set(pos)
    pos8 = pos16[:, :8]
    posT8 = jnp.zeros((8, NP), jnp.float32).at[:3, :N].set(pos.T)

    def pad_wp(w):
        return jnp.zeros((16, C), jnp.float32).at[:3].set(w)

    wgx = W_g[:C]
    wgp = jnp.zeros((16, C), jnp.float32).at[:3].set(W_g[C:C + 3])
    wo2 = jnp.zeros((C, 16), jnp.float32).at[:, :3].set(W_o2)
    bo2 = jnp.zeros((1, 16), jnp.float32).at[0, :3].set(b_o2)
    r1 = lambda b: b.reshape(1, C)

    weights = (
        W_lin0, r1(b_lin0), W_src0, r1(b_src0), W_dst0, r1(b_dst0),
        pad_wp(W_pos0), r1(b_pos0),
        W_lin1, r1(b_lin1), W_src1, r1(b_src1), W_dst1, r1(b_dst1),
        pad_wp(W_pos1), r1(b_pos1),
        wgx, wgp, r1(b_g), W_o1, r1(b_o1), wo2, bo2,
    )

    idxcat = _knn(pos8, posT8)
    idx_flat = idxcat.reshape(N * KC)

    xg, pg = _gather_sc(x, pos16, idx_flat)

    out16 = _conv(x, pos16, xg, pg, weights)
    return out16[:, :3]


# speedup vs baseline: 1.7918x; 1.7918x over previous
import functools

import jax
import jax.numpy as jnp
from jax import lax
from jax.experimental import pallas as pl
from jax.experimental.pallas import tpu as pltpu
from jax.experimental.pallas import tpu_sc as plsc

N = 10000
C = 128
K = 16
KC = 32
NP = 10240
RK = 80
RC = 200
INF = float("inf")


def _knn_body(pos8_ref, posT8_ref, out_ref, dist_ref, dm_ref):
    pid = pl.program_id(0)
    prow = pos8_ref[...]
    pcol = posT8_ref[...]
    d2r = jnp.sum(prow * prow, axis=1, keepdims=True)
    d2c = jnp.sum(pcol * pcol, axis=0, keepdims=True)
    cross = jnp.dot(prow, pcol, preferred_element_type=jnp.float32)
    dist = d2r + d2c - 2.0 * cross
    colio = lax.broadcasted_iota(jnp.int32, (RK, NP), 1)
    rowio = lax.broadcasted_iota(jnp.int32, (RK, NP), 0) + pid * RK
    dist_ref[...] = jnp.where((colio == rowio) | (colio >= N), INF, dist)

    nsub = NP // 128
    lane128 = lax.broadcasted_iota(jnp.int32, (RK, 128), 1)
    subio = lax.broadcasted_iota(jnp.int32, (RK, nsub, 128), 1)
    NR = KC - 1
    BIGCOL = 1 << 30

    def round_body(carry):
        T, cT, cnt, acc = carry
        d = dist_ref[...]
        dm2 = jnp.where((d < T) | ((d == T) & (colio <= cT)), INF, d)
        dm_ref[...] = dm2
        dm = dm_ref[...].reshape(RK, nsub, 128)

        def lane_top(excl):
            m = jnp.min(jnp.where(excl, INF, dm), axis=1)
            a = jnp.min(
                jnp.where((dm == m[:, None, :]) & ~excl, subio, nsub),
                axis=1)
            return m, a, excl | (subio == a[:, None, :])

        no = jnp.zeros((RK, nsub, 128), jnp.bool_)
        v1, a1, e1 = lane_top(no)
        v2, a2, e2 = lane_top(e1)
        v3, a3, e3 = lane_top(e2)
        v4, a4, _ = lane_top(e3)
        V = jnp.concatenate([v1, v2, v3, v4], axis=1)
        Cc = jnp.concatenate(
            [a1 * 128 + lane128, a2 * 128 + lane128,
             a3 * 128 + lane128, a4 * 128 + lane128], axis=1)
        r4 = lax.broadcasted_iota(jnp.int32, (RK, 512), 1) // 128
        enc = Cc * 4 + r4

        def mstep(t, st):
            T, cT, cnt, acc, active = st
            dead = (V < T) | ((V == T) & (Cc <= cT))
            veff = jnp.where(dead, INF, V)
            gm = jnp.min(veff, axis=1, keepdims=True)
            genc = jnp.min(jnp.where(veff == gm, enc, BIGCOL),
                           axis=1, keepdims=True)
            gi = genc // 4
            valid = (active > 0) & (cnt < NR) & (gm < INF)
            acc = jnp.where(valid & (cnt < K) & (lane128 == cnt), gi, acc)
            acc = jnp.where(
                valid & (cnt % 2 == 0) & (lane128 == K + cnt // 2), gi, acc)
            T = jnp.where(valid, gm, T)
            cT = jnp.where(valid, gi, cT)
            active = jnp.where(valid & (genc % 4 == 3), 0, active)
            cnt = cnt + jnp.where(valid, 1, 0)
            return T, cT, cnt, acc, active

        st = (T, cT, cnt, acc, jnp.ones((RK, 1), jnp.int32))
        st = lax.fori_loop(0, NR, mstep, st)
        return st[0], st[1], st[2], st[3]

    T0 = jnp.full((RK, 1), -INF, jnp.float32)
    c0 = jnp.full((RK, 1), -1, jnp.int32)
    carry = (T0, c0, jnp.zeros((RK, 1), jnp.int32),
             jnp.zeros((RK, 128), jnp.int32))
    carry = lax.while_loop(lambda c: jnp.min(c[2]) < NR, round_body, carry)
    out_ref[...] = carry[3][:, :KC]


def _knn(pos8, posT8):
    return pl.pallas_call(
        _knn_body,
        grid=(N // RK,),
        in_specs=[
            pl.BlockSpec((RK, 8), lambda i: (i, 0)),
            pl.BlockSpec((8, NP), lambda i: (0, 0)),
        ],
        out_specs=pl.BlockSpec((RK, KC), lambda i: (i, 0)),
        out_shape=jax.ShapeDtypeStruct((N, KC), jnp.int32),
        scratch_shapes=[pltpu.VMEM((RK, NP), jnp.float32),
                        pltpu.VMEM((RK, NP), jnp.float32)],
    )(pos8, posT8)


def _gather_sc(x, pos16, idx_flat):
    info = plsc.get_sparse_core_info()
    nc, ns = info.num_cores, info.num_subcores
    nw = nc * ns
    b_per_w = (N * KC) // nw
    ch = 128
    nfull = b_per_w // ch
    tail = b_per_w - nfull * ch

    mesh = plsc.VectorSubcoreMesh(core_axis_name="c", subcore_axis_name="s")

    @functools.partial(
        pl.kernel, mesh=mesh,
        compiler_params=pltpu.CompilerParams(use_tc_tiling_on_sc=False),
        out_type=(
            jax.ShapeDtypeStruct((N * KC, C), jnp.float32),
            jax.ShapeDtypeStruct((N * KC, 16), jnp.float32),
        ),
        scratch_types=[
            pltpu.VMEM((ch,), jnp.int32),
            pltpu.VMEM((ch, C), jnp.float32),
            pltpu.VMEM((ch, 16), jnp.float32),
            pltpu.SemaphoreType.DMA,
            pltpu.SemaphoreType.DMA,
        ],
    )
    def gk(x_hbm, p_hbm, idx_hbm, xg_hbm, pg_hbm, idx_v, xrow_v, prow_v,
           sem1, sem2):
        wid = lax.axis_index("s") * nc + lax.axis_index("c")
        base_w = wid * b_per_w

        def run_chunk(base, cnt):
            pltpu.sync_copy(idx_hbm.at[pl.ds(base, cnt)],
                            idx_v.at[pl.ds(0, cnt)])
            cp1 = pltpu.async_copy(x_hbm.at[idx_v.at[pl.ds(0, cnt)]],
                                   xrow_v.at[pl.ds(0, cnt)], sem1)
            cp2 = pltpu.async_copy(p_hbm.at[idx_v.at[pl.ds(0, cnt)]],
                                   prow_v.at[pl.ds(0, cnt)], sem2)
            cp1.wait()
            cp2.wait()
            pltpu.sync_copy(xrow_v.at[pl.ds(0, cnt)],
                            xg_hbm.at[pl.ds(base, cnt)])
            pltpu.sync_copy(prow_v.at[pl.ds(0, cnt)],
                            pg_hbm.at[pl.ds(base, cnt)])

        def body(c, carry):
            run_chunk(base_w + c * ch, ch)
            return carry

        lax.fori_loop(0, nfull, body, 0)
        run_chunk(base_w + nfull * ch, tail)

    return gk(x, pos16, idx_flat)


def _leaky(x):
    return jnp.where(x >= 0, x, 0.2 * x)


def _conv_body(x_ref, pos_ref, xg_ref, pg_ref,
               wl0_ref, bl0_ref, ws0_ref, bs0_ref, wd0_ref, bd0_ref,
               wp0_ref, bp0_ref,
               wl1_ref, bl1_ref, ws1_ref, bs1_ref, wd1_ref, bd1_ref,
               wp1_ref, bp1_ref,
               wgx_ref, wgp_ref, bg_ref, wo1_ref, bo1_ref, wo2_ref, bo2_ref,
               out_ref):
    xb = x_ref[...]
    posb = pos_ref[...]
    xg3 = xg_ref[...].reshape(RC, KC, C)
    pg3 = pg_ref[...].reshape(RC, KC, 16)

    h = _leaky(jnp.dot(xb, wgx_ref[...], preferred_element_type=jnp.float32)
               + jnp.dot(posb, wgp_ref[...],
                         preferred_element_type=jnp.float32)
               + bg_ref[...])

    layer = [
        (wl0_ref, bl0_ref, ws0_ref, bs0_ref, wd0_ref, bd0_ref, wp0_ref,
         bp0_ref),
        (wl1_ref, bl1_ref, ws1_ref, bs1_ref, wd1_ref, bd1_ref, wp1_ref,
         bp1_ref),
    ]
    for li, (wl, bl, ws, bs, wd, bd, wp, bp) in enumerate(layer):
        xgl = xg3[:, li * K:(li + 1) * K, :].reshape(RC * K, C)
        pgl = pg3[:, li * K:(li + 1) * K, :]
        v = jnp.dot(xgl, wl[...], preferred_element_type=jnp.float32) + bl[...]
        asrc = (jnp.dot(xgl, ws[...], preferred_element_type=jnp.float32)
                + bs[...])
        adst = (jnp.dot(xb, wd[...], preferred_element_type=jnp.float32)
                + bd[...])
        rel = (posb[:, None, :] - pgl).reshape(RC * K, 16)
        delta = (jnp.dot(rel, wp[...], preferred_element_type=jnp.float32)
                 + bp[...])
        delta3 = delta.reshape(RC, K, C)
        alpha = adst[:, None, :] - asrc.reshape(RC, K, C) + delta3
        mx = jnp.max(alpha, axis=1, keepdims=True)
        e = jnp.exp(alpha - mx)
        p = e / jnp.sum(e, axis=1, keepdims=True)
        msg = p * (v.reshape(RC, K, C) + delta3)
        h = h + jnp.sum(msg, axis=1)

    res = (jnp.dot(_leaky(jnp.dot(h, wo1_ref[...],
                                  preferred_element_type=jnp.float32)
                          + bo1_ref[...]),
                   wo2_ref[...], preferred_element_type=jnp.float32)
           + bo2_ref[...])
    out_ref[...] = res + posb


def _conv(x, pos16, xg, pg, weights):
    full = lambda a: pl.BlockSpec(a.shape, lambda i: tuple(0 for _ in a.shape))
    in_specs = [
        pl.BlockSpec((RC, C), lambda i: (i, 0)),
        pl.BlockSpec((RC, 16), lambda i: (i, 0)),
        pl.BlockSpec((RC * KC, C), lambda i: (i, 0)),
        pl.BlockSpec((RC * KC, 16), lambda i: (i, 0)),
    ] + [full(w) for w in weights]
    return pl.pallas_call(
        _conv_body,
        grid=(N // RC,),
        in_specs=in_specs,
        out_specs=pl.BlockSpec((RC, 16), lambda i: (i, 0)),
        out_shape=jax.ShapeDtypeStruct((N, 16), jnp.float32),
    )(x, pos16, xg, pg, *weights)


def kernel(x, pos,
           W_lin0, b_lin0, W_src0, b_src0, W_dst0, b_dst0, W_pos0, b_pos0,
           W_lin1, b_lin1, W_src1, b_src1, W_dst1, b_dst1, W_pos1, b_pos1,
           W_g, b_g, W_o1, b_o1, W_o2, b_o2):
    pos16 = jnp.zeros((N, 16), jnp.float32).at[:, :3].set(pos)
    pos8 = pos16[:, :8]
    posT8 = jnp.zeros((8, NP), jnp.float32).at[:3, :N].set(pos.T)

    def pad_wp(w):
        return jnp.zeros((16, C), jnp.float32).at[:3].set(w)

    wgx = W_g[:C]
    wgp = jnp.zeros((16, C), jnp.float32).at[:3].set(W_g[C:C + 3])
    wo2 = jnp.zeros((C, 16), jnp.float32).at[:, :3].set(W_o2)
    bo2 = jnp.zeros((1, 16), jnp.float32).at[0, :3].set(b_o2)
    r1 = lambda b: b.reshape(1, C)

    weights = (
        W_lin0, r1(b_lin0), W_src0, r1(b_src0), W_dst0, r1(b_dst0),
        pad_wp(W_pos0), r1(b_pos0),
        W_lin1, r1(b_lin1), W_src1, r1(b_src1), W_dst1, r1(b_dst1),
        pad_wp(W_pos1), r1(b_pos1),
        wgx, wgp, r1(b_g), W_o1, r1(b_o1), wo2, bo2,
    )

    idxcat = _knn(pos8, posT8)
    idx_flat = idxcat.reshape(N * KC)

    xg, pg = _gather_sc(x, pos16, idx_flat)

    out16 = _conv(x, pos16, xg, pg, weights)
    return out16[:, :3]


# speedup vs baseline: 2.0479x; 1.1429x over previous
import functools

import jax
import jax.numpy as jnp
from jax import lax
from jax.experimental import pallas as pl
from jax.experimental.pallas import tpu as pltpu
from jax.experimental.pallas import tpu_sc as plsc

N = 10000
C = 128
K = 16
KC = 32
NP = 10240
RK = 80
RC = 200
INF = float("inf")


def _knn_body(pos8_ref, posT8_ref, out_ref, dist_ref, dm_ref):
    pid = pl.program_id(0)
    prow = pos8_ref[...]
    pcol = posT8_ref[...]
    d2r = jnp.sum(prow * prow, axis=1, keepdims=True)
    d2c = jnp.sum(pcol * pcol, axis=0, keepdims=True)
    cross = jnp.dot(prow, pcol, preferred_element_type=jnp.float32)
    dist = d2r + d2c - 2.0 * cross
    colio = lax.broadcasted_iota(jnp.int32, (RK, NP), 1)
    rowio = lax.broadcasted_iota(jnp.int32, (RK, NP), 0) + pid * RK
    dist_ref[...] = jnp.where((colio == rowio) | (colio >= N), INF, dist)

    nsub = NP // 128
    lane128 = lax.broadcasted_iota(jnp.int32, (RK, 128), 1)
    subio = lax.broadcasted_iota(jnp.int32, (RK, nsub, 128), 1)
    NR = KC - 1
    BIGCOL = 1 << 30

    def round_body(carry):
        T, cT, cnt, acc = carry
        d = dist_ref[...]
        dm2 = jnp.where((d < T) | ((d == T) & (colio <= cT)), INF, d)
        dm_ref[...] = dm2
        dm = dm_ref[...].reshape(RK, nsub, 128)

        def lane_top(excl):
            m = jnp.min(jnp.where(excl, INF, dm), axis=1)
            a = jnp.min(
                jnp.where((dm == m[:, None, :]) & ~excl, subio, nsub),
                axis=1)
            return m, a, excl | (subio == a[:, None, :])

        NRANK = 6
        vs, cs = [], []
        excl = jnp.zeros((RK, nsub, 128), jnp.bool_)
        for _ in range(NRANK):
            m, a, excl = lane_top(excl)
            vs.append(m)
            cs.append(a * 128 + lane128)
        V0 = jnp.concatenate(vs, axis=1)
        Cc = jnp.concatenate(cs, axis=1)
        r6 = lax.broadcasted_iota(
            jnp.int32, (RK, NRANK * 128), 1) // 128
        enc = Cc * 8 + r6

        def mstep(t, st):
            V, T, cT, cnt, acc, active = st
            gm = jnp.min(V, axis=1, keepdims=True)
            genc = jnp.min(jnp.where(V == gm, enc, BIGCOL),
                           axis=1, keepdims=True)
            gi = genc // 8
            valid = (active > 0) & (cnt < NR) & (gm < INF)
            acc = jnp.where(valid & (cnt < K) & (lane128 == cnt), gi, acc)
            acc = jnp.where(
                valid & (cnt % 2 == 0) & (lane128 == K + cnt // 2), gi, acc)
            T = jnp.where(valid, gm, T)
            cT = jnp.where(valid, gi, cT)
            active = jnp.where(valid & (genc % 8 == NRANK - 1), 0, active)
            V = jnp.where(enc == genc, INF, V)
            cnt = cnt + jnp.where(valid, 1, 0)
            return V, T, cT, cnt, acc, active

        st = (V0, T, cT, cnt, acc, jnp.ones((RK, 1), jnp.int32))
        st = lax.fori_loop(0, NR, mstep, st)
        return st[1], st[2], st[3], st[4]

    T0 = jnp.full((RK, 1), -INF, jnp.float32)
    c0 = jnp.full((RK, 1), -1, jnp.int32)
    carry = (T0, c0, jnp.zeros((RK, 1), jnp.int32),
             jnp.zeros((RK, 128), jnp.int32))
    carry = lax.while_loop(lambda c: jnp.min(c[2]) < NR, round_body, carry)
    out_ref[...] = carry[3][:, :KC]


def _knn(pos8, posT8):
    return pl.pallas_call(
        _knn_body,
        grid=(N // RK,),
        in_specs=[
            pl.BlockSpec((RK, 8), lambda i: (i, 0)),
            pl.BlockSpec((8, NP), lambda i: (0, 0)),
        ],
        out_specs=pl.BlockSpec((RK, KC), lambda i: (i, 0)),
        out_shape=jax.ShapeDtypeStruct((N, KC), jnp.int32),
        scratch_shapes=[pltpu.VMEM((RK, NP), jnp.float32),
                        pltpu.VMEM((RK, NP), jnp.float32)],
    )(pos8, posT8)


def _gather_sc(x, pos16, idx_flat):
    info = plsc.get_sparse_core_info()
    nc, ns = info.num_cores, info.num_subcores
    nw = nc * ns
    b_per_w = (N * KC) // nw
    ch = 128
    nfull = b_per_w // ch
    tail = b_per_w - nfull * ch

    mesh = plsc.VectorSubcoreMesh(core_axis_name="c", subcore_axis_name="s")

    @functools.partial(
        pl.kernel, mesh=mesh,
        compiler_params=pltpu.CompilerParams(use_tc_tiling_on_sc=False),
        out_type=(
            jax.ShapeDtypeStruct((N * KC, C), jnp.float32),
            jax.ShapeDtypeStruct((N * KC, 16), jnp.float32),
        ),
        scratch_types=[
            pltpu.VMEM((ch,), jnp.int32),
            pltpu.VMEM((ch, C), jnp.float32),
            pltpu.VMEM((ch, 16), jnp.float32),
            pltpu.SemaphoreType.DMA,
            pltpu.SemaphoreType.DMA,
        ],
    )
    def gk(x_hbm, p_hbm, idx_hbm, xg_hbm, pg_hbm, idx_v, xrow_v, prow_v,
           sem1, sem2):
        wid = lax.axis_index("s") * nc + lax.axis_index("c")
        base_w = wid * b_per_w

        def run_chunk(base, cnt):
            pltpu.sync_copy(idx_hbm.at[pl.ds(base, cnt)],
                            idx_v.at[pl.ds(0, cnt)])
            cp1 = pltpu.async_copy(x_hbm.at[idx_v.at[pl.ds(0, cnt)]],
                                   xrow_v.at[pl.ds(0, cnt)], sem1)
            cp2 = pltpu.async_copy(p_hbm.at[idx_v.at[pl.ds(0, cnt)]],
                                   prow_v.at[pl.ds(0, cnt)], sem2)
            cp1.wait()
            cp2.wait()
            pltpu.sync_copy(xrow_v.at[pl.ds(0, cnt)],
                            xg_hbm.at[pl.ds(base, cnt)])
            pltpu.sync_copy(prow_v.at[pl.ds(0, cnt)],
                            pg_hbm.at[pl.ds(base, cnt)])

        def body(c, carry):
            run_chunk(base_w + c * ch, ch)
            return carry

        lax.fori_loop(0, nfull, body, 0)
        run_chunk(base_w + nfull * ch, tail)

    return gk(x, pos16, idx_flat)


def _leaky(x):
    return jnp.where(x >= 0, x, 0.2 * x)


def _conv_body(x_ref, pos_ref, xg_ref, pg_ref,
               wl0_ref, bl0_ref, ws0_ref, bs0_ref, wd0_ref, bd0_ref,
               wp0_ref, bp0_ref,
               wl1_ref, bl1_ref, ws1_ref, bs1_ref, wd1_ref, bd1_ref,
               wp1_ref, bp1_ref,
               wgx_ref, wgp_ref, bg_ref, wo1_ref, bo1_ref, wo2_ref, bo2_ref,
               out_ref):
    xb = x_ref[...]
    posb = pos_ref[...]
    xg3 = xg_ref[...].reshape(RC, KC, C)
    pg3 = pg_ref[...].reshape(RC, KC, 16)

    h = _leaky(jnp.dot(xb, wgx_ref[...], preferred_element_type=jnp.float32)
               + jnp.dot(posb, wgp_ref[...],
                         preferred_element_type=jnp.float32)
               + bg_ref[...])

    layer = [
        (wl0_ref, bl0_ref, ws0_ref, bs0_ref, wd0_ref, bd0_ref, wp0_ref,
         bp0_ref),
        (wl1_ref, bl1_ref, ws1_ref, bs1_ref, wd1_ref, bd1_ref, wp1_ref,
         bp1_ref),
    ]
    for li, (wl, bl, ws, bs, wd, bd, wp, bp) in enumerate(layer):
        xgl = xg3[:, li * K:(li + 1) * K, :].reshape(RC * K, C)
        pgl = pg3[:, li * K:(li + 1) * K, :]
        v = jnp.dot(xgl, wl[...], preferred_element_type=jnp.float32) + bl[...]
        asrc = (jnp.dot(xgl, ws[...], preferred_element_type=jnp.float32)
                + bs[...])
        adst = (jnp.dot(xb, wd[...], preferred_element_type=jnp.float32)
                + bd[...])
        rel = (posb[:, None, :] - pgl).reshape(RC * K, 16)
        delta = (jnp.dot(rel, wp[...], preferred_element_type=jnp.float32)
                 + bp[...])
        delta3 = delta.reshape(RC, K, C)
        alpha = adst[:, None, :] - asrc.reshape(RC, K, C) + delta3
        mx = jnp.max(alpha, axis=1, keepdims=True)
        e = jnp.exp(alpha - mx)
        p = e / jnp.sum(e, axis=1, keepdims=True)
        msg = p * (v.reshape(RC, K, C) + delta3)
        h = h + jnp.sum(msg, axis=1)

    res = (jnp.dot(_leaky(jnp.dot(h, wo1_ref[...],
                                  preferred_element_type=jnp.float32)
                          + bo1_ref[...]),
                   wo2_ref[...], preferred_element_type=jnp.float32)
           + bo2_ref[...])
    out_ref[...] = res + posb


def _conv(x, pos16, xg, pg, weights):
    full = lambda a: pl.BlockSpec(a.shape, lambda i: tuple(0 for _ in a.shape))
    in_specs = [
        pl.BlockSpec((RC, C), lambda i: (i, 0)),
        pl.BlockSpec((RC, 16), lambda i: (i, 0)),
        pl.BlockSpec((RC * KC, C), lambda i: (i, 0)),
        pl.BlockSpec((RC * KC, 16), lambda i: (i, 0)),
    ] + [full(w) for w in weights]
    return pl.pallas_call(
        _conv_body,
        grid=(N // RC,),
        in_specs=in_specs,
        out_specs=pl.BlockSpec((RC, 16), lambda i: (i, 0)),
        out_shape=jax.ShapeDtypeStruct((N, 16), jnp.float32),
    )(x, pos16, xg, pg, *weights)


def kernel(x, pos,
           W_lin0, b_lin0, W_src0, b_src0, W_dst0, b_dst0, W_pos0, b_pos0,
           W_lin1, b_lin1, W_src1, b_src1, W_dst1, b_dst1, W_pos1, b_pos1,
           W_g, b_g, W_o1, b_o1, W_o2, b_o2):
    pos16 = jnp.zeros((N, 16), jnp.float32).at[:, :3].set(pos)
    pos8 = pos16[:, :8]
    posT8 = jnp.zeros((8, NP), jnp.float32).at[:3, :N].set(pos.T)

    def pad_wp(w):
        return jnp.zeros((16, C), jnp.float32).at[:3].set(w)

    wgx = W_g[:C]
    wgp = jnp.zeros((16, C), jnp.float32).at[:3].set(W_g[C:C + 3])
    wo2 = jnp.zeros((C, 16), jnp.float32).at[:, :3].set(W_o2)
    bo2 = jnp.zeros((1, 16), jnp.float32).at[0, :3].set(b_o2)
    r1 = lambda b: b.reshape(1, C)

    weights = (
        W_lin0, r1(b_lin0), W_src0, r1(b_src0), W_dst0, r1(b_dst0),
        pad_wp(W_pos0), r1(b_pos0),
        W_lin1, r1(b_lin1), W_src1, r1(b_src1), W_dst1, r1(b_dst1),
        pad_wp(W_pos1), r1(b_pos1),
        wgx, wgp, r1(b_g), W_o1, r1(b_o1), wo2, bo2,
    )

    idxcat = _knn(pos8, posT8)
    idx_flat = idxcat.reshape(N * KC)

    xg, pg = _gather_sc(x, pos16, idx_flat)

    out16 = _conv(x, pos16, xg, pg, weights)
    return out16[:, :3]


# speedup vs baseline: 2.5486x; 1.2445x over previous
import functools

import jax
import jax.numpy as jnp
from jax import lax
from jax.experimental import pallas as pl
from jax.experimental.pallas import tpu as pltpu
from jax.experimental.pallas import tpu_sc as plsc

N = 10000
C = 128
K = 16
KC = 32
NP = 10240
RK = 80
RC = 200
INF = float("inf")


def _knn_body(pos8_ref, posT8_ref, out_ref, dist_ref, dm_ref):
    pid = pl.program_id(0)
    prow = pos8_ref[...]
    pcol = posT8_ref[...]
    d2r = jnp.sum(prow * prow, axis=1, keepdims=True)
    d2c = jnp.sum(pcol * pcol, axis=0, keepdims=True)
    cross = jnp.dot(prow, pcol, preferred_element_type=jnp.float32)
    dist = d2r + d2c - 2.0 * cross
    colio = lax.broadcasted_iota(jnp.int32, (RK, NP), 1)
    rowio = lax.broadcasted_iota(jnp.int32, (RK, NP), 0) + pid * RK
    dist_ref[...] = jnp.where((colio == rowio) | (colio >= N), INF, dist)

    nsub = NP // 128
    lane128 = lax.broadcasted_iota(jnp.int32, (RK, 128), 1)
    subio = lax.broadcasted_iota(jnp.int32, (RK, nsub, 128), 1)
    NR = KC - 1
    BIGCOL = 1 << 30

    def round_body(carry):
        T, cT, cnt, acc = carry
        d = dist_ref[...]
        dm2 = jnp.where((d < T) | ((d == T) & (colio <= cT)), INF, d)
        dm_ref[...] = dm2
        dm = dm_ref[...].reshape(RK, nsub, 128)

        def lane_top(excl):
            m = jnp.min(jnp.where(excl, INF, dm), axis=1)
            a = jnp.min(
                jnp.where((dm == m[:, None, :]) & ~excl, subio, nsub),
                axis=1)
            return m, a, excl | (subio == a[:, None, :])

        NRANK = 6
        vs, cs = [], []
        excl = jnp.zeros((RK, nsub, 128), jnp.bool_)
        for _ in range(NRANK):
            m, a, excl = lane_top(excl)
            vs.append(m)
            cs.append(a * 128 + lane128)
        V0 = jnp.concatenate(vs, axis=1)
        Cc = jnp.concatenate(cs, axis=1)
        r6 = lax.broadcasted_iota(
            jnp.int32, (RK, NRANK * 128), 1) // 128
        enc = Cc * 8 + r6

        def mstep(t, st):
            V, T, cT, cnt, acc, active = st
            gm = jnp.min(V, axis=1, keepdims=True)
            genc = jnp.min(jnp.where(V == gm, enc, BIGCOL),
                           axis=1, keepdims=True)
            gi = genc // 8
            valid = (active > 0) & (cnt < NR) & (gm < INF)
            acc = jnp.where(valid & (cnt < K) & (lane128 == cnt), gi, acc)
            acc = jnp.where(
                valid & (cnt % 2 == 0) & (lane128 == K + cnt // 2), gi, acc)
            T = jnp.where(valid, gm, T)
            cT = jnp.where(valid, gi, cT)
            active = jnp.where(valid & (genc % 8 == NRANK - 1), 0, active)
            V = jnp.where(enc == genc, INF, V)
            cnt = cnt + jnp.where(valid, 1, 0)
            return V, T, cT, cnt, acc, active

        st = (V0, T, cT, cnt, acc, jnp.ones((RK, 1), jnp.int32))
        for t in range(NR):
            st = mstep(t, st)
        return st[1], st[2], st[3], st[4]

    T0 = jnp.full((RK, 1), -INF, jnp.float32)
    c0 = jnp.full((RK, 1), -1, jnp.int32)
    carry = (T0, c0, jnp.zeros((RK, 1), jnp.int32),
             jnp.zeros((RK, 128), jnp.int32))
    carry = lax.while_loop(lambda c: jnp.min(c[2]) < NR, round_body, carry)
    out_ref[...] = carry[3][:, :KC]


def _knn(pos8, posT8):
    return pl.pallas_call(
        _knn_body,
        grid=(N // RK,),
        in_specs=[
            pl.BlockSpec((RK, 8), lambda i: (i, 0)),
            pl.BlockSpec((8, NP), lambda i: (0, 0)),
        ],
        out_specs=pl.BlockSpec((RK, KC), lambda i: (i, 0)),
        out_shape=jax.ShapeDtypeStruct((N, KC), jnp.int32),
        scratch_shapes=[pltpu.VMEM((RK, NP), jnp.float32),
                        pltpu.VMEM((RK, NP), jnp.float32)],
    )(pos8, posT8)


def _gather_sc(x, pos16, idx_flat):
    info = plsc.get_sparse_core_info()
    nc, ns = info.num_cores, info.num_subcores
    nw = nc * ns
    b_per_w = (N * KC) // nw
    ch = 128
    nfull = b_per_w // ch
    tail = b_per_w - nfull * ch

    mesh = plsc.VectorSubcoreMesh(core_axis_name="c", subcore_axis_name="s")

    @functools.partial(
        pl.kernel, mesh=mesh,
        compiler_params=pltpu.CompilerParams(use_tc_tiling_on_sc=False),
        out_type=(
            jax.ShapeDtypeStruct((N * KC, C), jnp.float32),
            jax.ShapeDtypeStruct((N * KC, 16), jnp.float32),
        ),
        scratch_types=[
            pltpu.VMEM((ch,), jnp.int32),
            pltpu.VMEM((ch, C), jnp.float32),
            pltpu.VMEM((ch, 16), jnp.float32),
            pltpu.SemaphoreType.DMA,
            pltpu.SemaphoreType.DMA,
        ],
    )
    def gk(x_hbm, p_hbm, idx_hbm, xg_hbm, pg_hbm, idx_v, xrow_v, prow_v,
           sem1, sem2):
        wid = lax.axis_index("s") * nc + lax.axis_index("c")
        base_w = wid * b_per_w

        def run_chunk(base, cnt):
            pltpu.sync_copy(idx_hbm.at[pl.ds(base, cnt)],
                            idx_v.at[pl.ds(0, cnt)])
            cp1 = pltpu.async_copy(x_hbm.at[idx_v.at[pl.ds(0, cnt)]],
                                   xrow_v.at[pl.ds(0, cnt)], sem1)
            cp2 = pltpu.async_copy(p_hbm.at[idx_v.at[pl.ds(0, cnt)]],
                                   prow_v.at[pl.ds(0, cnt)], sem2)
            cp1.wait()
            cp2.wait()
            pltpu.sync_copy(xrow_v.at[pl.ds(0, cnt)],
                            xg_hbm.at[pl.ds(base, cnt)])
            pltpu.sync_copy(prow_v.at[pl.ds(0, cnt)],
                            pg_hbm.at[pl.ds(base, cnt)])

        def body(c, carry):
            run_chunk(base_w + c * ch, ch)
            return carry

        lax.fori_loop(0, nfull, body, 0)
        run_chunk(base_w + nfull * ch, tail)

    return gk(x, pos16, idx_flat)


def _leaky(x):
    return jnp.where(x >= 0, x, 0.2 * x)


def _conv_body(x_ref, pos_ref, xg_ref, pg_ref,
               wl0_ref, bl0_ref, ws0_ref, bs0_ref, wd0_ref, bd0_ref,
               wp0_ref, bp0_ref,
               wl1_ref, bl1_ref, ws1_ref, bs1_ref, wd1_ref, bd1_ref,
               wp1_ref, bp1_ref,
               wgx_ref, wgp_ref, bg_ref, wo1_ref, bo1_ref, wo2_ref, bo2_ref,
               out_ref):
    xb = x_ref[...]
    posb = pos_ref[...]
    xg3 = xg_ref[...].reshape(RC, KC, C)
    pg3 = pg_ref[...].reshape(RC, KC, 16)

    h = _leaky(jnp.dot(xb, wgx_ref[...], preferred_element_type=jnp.float32)
               + jnp.dot(posb, wgp_ref[...],
                         preferred_element_type=jnp.float32)
               + bg_ref[...])

    layer = [
        (wl0_ref, bl0_ref, ws0_ref, bs0_ref, wd0_ref, bd0_ref, wp0_ref,
         bp0_ref),
        (wl1_ref, bl1_ref, ws1_ref, bs1_ref, wd1_ref, bd1_ref, wp1_ref,
         bp1_ref),
    ]
    for li, (wl, bl, ws, bs, wd, bd, wp, bp) in enumerate(layer):
        xgl = xg3[:, li * K:(li + 1) * K, :].reshape(RC * K, C)
        pgl = pg3[:, li * K:(li + 1) * K, :]
        v = jnp.dot(xgl, wl[...], preferred_element_type=jnp.float32) + bl[...]
        asrc = (jnp.dot(xgl, ws[...], preferred_element_type=jnp.float32)
                + bs[...])
        adst = (jnp.dot(xb, wd[...], preferred_element_type=jnp.float32)
                + bd[...])
        rel = (posb[:, None, :] - pgl).reshape(RC * K, 16)
        delta = (jnp.dot(rel, wp[...], preferred_element_type=jnp.float32)
                 + bp[...])
        delta3 = delta.reshape(RC, K, C)
        alpha = adst[:, None, :] - asrc.reshape(RC, K, C) + delta3
        mx = jnp.max(alpha, axis=1, keepdims=True)
        e = jnp.exp(alpha - mx)
        p = e / jnp.sum(e, axis=1, keepdims=True)
        msg = p * (v.reshape(RC, K, C) + delta3)
        h = h + jnp.sum(msg, axis=1)

    res = (jnp.dot(_leaky(jnp.dot(h, wo1_ref[...],
                                  preferred_element_type=jnp.float32)
                          + bo1_ref[...]),
                   wo2_ref[...], preferred_element_type=jnp.float32)
           + bo2_ref[...])
    out_ref[...] = res + posb


def _conv(x, pos16, xg, pg, weights):
    full = lambda a: pl.BlockSpec(a.shape, lambda i: tuple(0 for _ in a.shape))
    in_specs = [
        pl.BlockSpec((RC, C), lambda i: (i, 0)),
        pl.BlockSpec((RC, 16), lambda i: (i, 0)),
        pl.BlockSpec((RC * KC, C), lambda i: (i, 0)),
        pl.BlockSpec((RC * KC, 16), lambda i: (i, 0)),
    ] + [full(w) for w in weights]
    return pl.pallas_call(
        _conv_body,
        grid=(N // RC,),
        in_specs=in_specs,
        out_specs=pl.BlockSpec((RC, 16), lambda i: (i, 0)),
        out_shape=jax.ShapeDtypeStruct((N, 16), jnp.float32),
    )(x, pos16, xg, pg, *weights)


def kernel(x, pos,
           W_lin0, b_lin0, W_src0, b_src0, W_dst0, b_dst0, W_pos0, b_pos0,
           W_lin1, b_lin1, W_src1, b_src1, W_dst1, b_dst1, W_pos1, b_pos1,
           W_g, b_g, W_o1, b_o1, W_o2, b_o2):
    pos16 = jnp.zeros((N, 16), jnp.float32).at[:, :3].set(pos)
    pos8 = pos16[:, :8]
    posT8 = jnp.zeros((8, NP), jnp.float32).at[:3, :N].set(pos.T)

    def pad_wp(w):
        return jnp.zeros((16, C), jnp.float32).at[:3].set(w)

    wgx = W_g[:C]
    wgp = jnp.zeros((16, C), jnp.float32).at[:3].set(W_g[C:C + 3])
    wo2 = jnp.zeros((C, 16), jnp.float32).at[:, :3].set(W_o2)
    bo2 = jnp.zeros((1, 16), jnp.float32).at[0, :3].set(b_o2)
    r1 = lambda b: b.reshape(1, C)

    weights = (
        W_lin0, r1(b_lin0), W_src0, r1(b_src0), W_dst0, r1(b_dst0),
        pad_wp(W_pos0), r1(b_pos0),
        W_lin1, r1(b_lin1), W_src1, r1(b_src1), W_dst1, r1(b_dst1),
        pad_wp(W_pos1), r1(b_pos1),
        wgx, wgp, r1(b_g), W_o1, r1(b_o1), wo2, bo2,
    )

    idxcat = _knn(pos8, posT8)
    idx_flat = idxcat.reshape(N * KC)

    xg, pg = _gather_sc(x, pos16, idx_flat)

    out16 = _conv(x, pos16, xg, pg, weights)
    return out16[:, :3]
